# Initial kernel scaffold; baseline (speedup 1.0000x reference)
import jax
import jax.numpy as jnp
from jax.experimental import pallas as pl


def kernel(x, W1, b1, W2, b2, W3, b3):
    raise NotImplementedError("write your pallas kernel here")


# speedup vs baseline: 14.3589x; 14.3589x over previous
import functools

import jax
import jax.numpy as jnp
from jax import lax
from jax.experimental import pallas as pl
from jax.experimental.pallas import tpu as pltpu
from jax.experimental.pallas import tpu_sc as plsc

_B = 8
_N = 4096
_C = 35
_K = 512
_S = 32
_R2 = 0.25 * 0.25
_CP = 128
_KC = 128
_NW = 32
_CH = 128


def _fps_body(px_ref, py_ref, pz_ref, ocx_ref, ocy_ref, ocz_ref):
    px = px_ref[...]
    py = py_ref[...]
    pz = pz_ref[...]
    lane_n = lax.broadcasted_iota(jnp.int32, (_B, _N), 1)
    lane_k = lax.broadcasted_iota(jnp.int32, (_B, _K), 1)

    lx = px[:, 0:1]
    ly = py[:, 0:1]
    lz = pz[:, 0:1]
    zero_k = jnp.zeros((_B, _K), jnp.float32)
    acx = jnp.where(lane_k == 0, lx, zero_k)
    acy = jnp.where(lane_k == 0, ly, zero_k)
    acz = jnp.where(lane_k == 0, lz, zero_k)
    dists = jnp.full((_B, _N), jnp.inf, jnp.float32)

    def body(i, st):
        dists, lx, ly, lz, acx, acy, acz = st
        d = (px - lx) ** 2 + (py - ly) ** 2 + (pz - lz) ** 2
        dists = jnp.minimum(dists, d)
        nxt = jnp.argmax(dists, axis=1).astype(jnp.int32)
        onehot = lane_n == nxt[:, None]
        lx = jnp.sum(jnp.where(onehot, px, 0.0), axis=1, keepdims=True)
        ly = jnp.sum(jnp.where(onehot, py, 0.0), axis=1, keepdims=True)
        lz = jnp.sum(jnp.where(onehot, pz, 0.0), axis=1, keepdims=True)
        sel = lane_k == i
        acx = jnp.where(sel, lx, acx)
        acy = jnp.where(sel, ly, acy)
        acz = jnp.where(sel, lz, acz)
        return dists, lx, ly, lz, acx, acy, acz

    st = (dists, lx, ly, lz, acx, acy, acz)
    st = lax.fori_loop(1, _K, body, st)
    _, _, _, _, acx, acy, acz = st
    ocx_ref[...] = acx
    ocy_ref[...] = acy
    ocz_ref[...] = acz


def _fps(px, py, pz):
    out = [jax.ShapeDtypeStruct((_B, _K), jnp.float32)] * 3
    return pl.pallas_call(_fps_body, out_shape=out)(px, py, pz)


def _ball_body(px_ref, py_ref, pz_ref, cx_ref, cy_ref, cz_ref, o_ref):
    b = pl.program_id(0)
    kb = pl.program_id(1)
    rsel = lax.broadcasted_iota(jnp.int32, (_B, _N), 0) == b
    pick_row = lambda r: jnp.sum(jnp.where(rsel, r[...], 0.0), axis=0,
                                 keepdims=True)
    pxr = pick_row(px_ref)
    pyr = pick_row(py_ref)
    pzr = pick_row(pz_ref)
    k0 = kb * _KC
    csel = lax.broadcasted_iota(jnp.int32, (_KC, _B), 1) == b
    pick_col = lambda r: jnp.sum(
        jnp.where(csel, r[pl.ds(k0, _KC), :], 0.0), axis=1, keepdims=True)
    cxc = pick_col(cx_ref)
    cyc = pick_col(cy_ref)
    czc = pick_col(cz_ref)

    cn = cxc * cxc + cyc * cyc + czc * czc
    pn = pxr * pxr + pyr * pyr + pzr * pzr
    cmat = jnp.concatenate(
        [cxc, cyc, czc, jnp.zeros((_KC, 5), jnp.float32)], axis=1)
    pmat = jnp.concatenate(
        [pxr, pyr, pzr, jnp.zeros((5, _N), jnp.float32)], axis=0)
    dot = jnp.dot(cmat, pmat, preferred_element_type=jnp.float32)
    d2 = cn + pn - 2.0 * dot
    mask = d2 < _R2

    iota = lax.broadcasted_iota(jnp.int32, (_KC, _N), 1)
    big = jnp.int32(_N)
    vals = jnp.where(mask, iota, big)

    cols = []
    first = None
    for s in range(_S):
        m = jnp.min(vals, axis=1, keepdims=True)
        found = m < big
        if s == 0:
            first = jnp.where(found, m, 0)
        cols.append(jnp.where(found, m, first))
        vals = jnp.where(vals == m, big, vals)
    idx = jnp.concatenate(cols, axis=1)
    o_ref[...] = (idx + b * _N)[None]


def _ball(px, py, pz, cxt, cyt, czt):
    full2 = lambda shape: pl.BlockSpec(shape, lambda b, k: (0, 0))
    return pl.pallas_call(
        _ball_body,
        grid=(_B, _K // _KC),
        in_specs=[
            full2((_B, _N)), full2((_B, _N)), full2((_B, _N)),
            full2((_K, _B)), full2((_K, _B)), full2((_K, _B)),
        ],
        out_specs=pl.BlockSpec((1, _KC, _S), lambda b, k: (b, k, 0)),
        out_shape=jax.ShapeDtypeStruct((_B, _K, _S), jnp.int32),
    )(px, py, pz, cxt, cyt, czt)


def _sc_gather(table, idx):
    tot = idx.shape[0]
    per_w = tot // _NW
    nch = per_w // _CH
    mesh = plsc.VectorSubcoreMesh(core_axis_name="c", subcore_axis_name="s")

    @functools.partial(
        pl.kernel,
        mesh=mesh,
        out_type=jax.ShapeDtypeStruct((tot, _CP), jnp.float32),
        scratch_types=[
            pltpu.VMEM((_CH,), jnp.int32),
            pltpu.VMEM((_CH, _CP), jnp.float32),
            pltpu.SemaphoreType.DMA,
        ],
    )
    def gk(table_hbm, idx_hbm, out_hbm, idx_v, rows_v, sem):
        wid = lax.axis_index("s") * 2 + lax.axis_index("c")
        w0 = wid * per_w

        def body(ch, carry):
            base = pl.multiple_of(w0 + ch * _CH, _CH)
            pltpu.sync_copy(idx_hbm.at[pl.ds(base, _CH)], idx_v)
            pltpu.async_copy(table_hbm.at[idx_v], rows_v, sem).wait()
            pltpu.sync_copy(rows_v, out_hbm.at[pl.ds(base, _CH)])
            return carry

        lax.fori_loop(0, nch, body, 0)

    return gk(table, idx)


def _mlp_body(g_ref, c_ref, w1_ref, wc_ref, w2_ref, b2_ref, w3_ref,
              b3_ref, o_ref):
    f32 = jnp.float32
    bias = jnp.dot(c_ref[...], wc_ref[...], preferred_element_type=f32)
    h = jnp.dot(g_ref[...], w1_ref[...], preferred_element_type=f32)
    h = h.reshape(_KC, _S, 128) + bias[:, None, :]
    h = jnp.maximum(h, 0.0).reshape(_KC * _S, 128)
    h = jnp.dot(h, w2_ref[...], preferred_element_type=f32) + b2_ref[0:1, :]
    h = jnp.maximum(h, 0.0)
    h = jnp.dot(h, w3_ref[...], preferred_element_type=f32) + b3_ref[0:1, :]
    h = jnp.maximum(h, 0.0)
    o_ref[...] = jnp.max(h.reshape(_KC, _S, 256), axis=1)


def _mlp(g, caug, w1p, wc, w2, b2t, w3, b3t):
    rows = _KC * _S
    nblk = (_B * _K) // _KC
    whole = lambda shape: pl.BlockSpec(shape, lambda i: (0, 0))
    return pl.pallas_call(
        _mlp_body,
        grid=(nblk,),
        in_specs=[
            pl.BlockSpec((rows, _CP), lambda i: (i, 0)),
            pl.BlockSpec((_KC, 8), lambda i: (i, 0)),
            whole((_CP, 128)),
            whole((8, 128)),
            whole((128, 256)),
            whole((8, 256)),
            whole((256, 256)),
            whole((8, 256)),
        ],
        out_specs=pl.BlockSpec((_KC, 256), lambda i: (i, 0)),
        out_shape=jax.ShapeDtypeStruct((_B * _K, 256), jnp.float32),
    )(g, caug, w1p, wc, w2, b2t, w3, b3t)


def kernel(x, W1, b1, W2, b2, W3, b3):
    px = x[:, :, 0]
    py = x[:, :, 1]
    pz = x[:, :, 2]

    cx, cy, cz = _fps(px, py, pz)

    idx = _ball(px, py, pz, cx.T, cy.T, cz.T)

    xpad = jnp.pad(x, ((0, 0), (0, 0), (0, _CP - _C)))
    table = xpad.reshape(_B * _N, _CP)
    g = _sc_gather(table, idx.reshape(-1))

    ones = jnp.ones((_B * _K, 1), jnp.float32)
    zeros4 = jnp.zeros((_B * _K, 4), jnp.float32)
    caug = jnp.concatenate(
        [cx.reshape(-1, 1), cy.reshape(-1, 1), cz.reshape(-1, 1), ones,
         zeros4], axis=1)
    w1p = jnp.concatenate([W1, jnp.zeros((_CP - _C, 128), jnp.float32)],
                          axis=0)
    wc = jnp.concatenate(
        [-W1[:3], b1[None, :], jnp.zeros((4, 128), jnp.float32)], axis=0)
    b2t = jnp.broadcast_to(b2[None, :], (8, 256))
    b3t = jnp.broadcast_to(b3[None, :], (8, 256))

    h = _mlp(g, caug, w1p, wc, W2, b2t, W3, b3t)

    centroid = jnp.stack([cx, cy, cz], axis=-1)
    return jnp.concatenate(
        [centroid, h.reshape(_B, _K, 256)], axis=2)


# speedup vs baseline: 17.7020x; 1.2328x over previous
import functools

import jax
import jax.numpy as jnp
from jax import lax
from jax.experimental import pallas as pl
from jax.experimental.pallas import tpu as pltpu
from jax.experimental.pallas import tpu_sc as plsc

_B = 8
_N = 4096
_C = 35
_K = 512
_S = 32
_R2 = 0.25 * 0.25
_CP = 128
_KC = 128
_NW = 32
_CH = 128


def _fps_body(px_ref, py_ref, pz_ref, ocx_ref, ocy_ref, ocz_ref):
    px = px_ref[...]
    py = py_ref[...]
    pz = pz_ref[...]
    lane_n = lax.broadcasted_iota(jnp.int32, (_B, _N), 1)
    lane_k = lax.broadcasted_iota(jnp.int32, (_B, _K), 1)

    lx = px[:, 0:1]
    ly = py[:, 0:1]
    lz = pz[:, 0:1]
    zero_k = jnp.zeros((_B, _K), jnp.float32)
    acx = jnp.where(lane_k == 0, lx, zero_k)
    acy = jnp.where(lane_k == 0, ly, zero_k)
    acz = jnp.where(lane_k == 0, lz, zero_k)
    dists = jnp.full((_B, _N), jnp.inf, jnp.float32)

    def body(i, st):
        dists, lx, ly, lz, acx, acy, acz = st
        d = (px - lx) ** 2 + (py - ly) ** 2 + (pz - lz) ** 2
        dists = jnp.minimum(dists, d)
        nxt = jnp.argmax(dists, axis=1).astype(jnp.int32)
        onehot = lane_n == nxt[:, None]
        lx = jnp.sum(jnp.where(onehot, px, 0.0), axis=1, keepdims=True)
        ly = jnp.sum(jnp.where(onehot, py, 0.0), axis=1, keepdims=True)
        lz = jnp.sum(jnp.where(onehot, pz, 0.0), axis=1, keepdims=True)
        sel = lane_k == i
        acx = jnp.where(sel, lx, acx)
        acy = jnp.where(sel, ly, acy)
        acz = jnp.where(sel, lz, acz)
        return dists, lx, ly, lz, acx, acy, acz

    st = (dists, lx, ly, lz, acx, acy, acz)
    st = lax.fori_loop(1, _K, body, st)
    _, _, _, _, acx, acy, acz = st
    ocx_ref[...] = acx
    ocy_ref[...] = acy
    ocz_ref[...] = acz


def _fps(px, py, pz):
    out = [jax.ShapeDtypeStruct((_B, _K), jnp.float32)] * 3
    return pl.pallas_call(_fps_body, out_shape=out)(px, py, pz)


def _ball_body(px_ref, py_ref, pz_ref, cx_ref, cy_ref, cz_ref, o_ref):
    b = pl.program_id(0)
    kb = pl.program_id(1)
    rsel = lax.broadcasted_iota(jnp.int32, (_B, _N), 0) == b
    pick_row = lambda r: jnp.sum(jnp.where(rsel, r[...], 0.0), axis=0,
                                 keepdims=True)
    pxr = pick_row(px_ref)
    pyr = pick_row(py_ref)
    pzr = pick_row(pz_ref)
    k0 = kb * _KC
    csel = lax.broadcasted_iota(jnp.int32, (_KC, _B), 1) == b
    pick_col = lambda r: jnp.sum(
        jnp.where(csel, r[pl.ds(k0, _KC), :], 0.0), axis=1, keepdims=True)
    cxc = pick_col(cx_ref)
    cyc = pick_col(cy_ref)
    czc = pick_col(cz_ref)

    cn = cxc * cxc + cyc * cyc + czc * czc
    pn = pxr * pxr + pyr * pyr + pzr * pzr
    cmat = jnp.concatenate(
        [cxc, cyc, czc, jnp.zeros((_KC, 5), jnp.float32)], axis=1)
    pmat = jnp.concatenate(
        [pxr, pyr, pzr, jnp.zeros((5, _N), jnp.float32)], axis=0)
    dot = jnp.dot(cmat, pmat, preferred_element_type=jnp.float32)
    d2 = cn + pn - 2.0 * dot
    mask = d2 < _R2

    iota_f = lax.broadcasted_iota(jnp.int32, (_KC, _N), 1).astype(jnp.float32)
    bigf = jnp.float32(_N)
    vals = jnp.where(mask, iota_f, bigf)

    cols = []
    first = None
    m = jnp.full((_KC, 1), -1.0, jnp.float32)
    for s in range(_S):
        masked = jnp.where(vals > m, vals, bigf)
        m = jnp.min(masked, axis=1, keepdims=True)
        found = m < bigf
        if s == 0:
            first = jnp.where(found, m, 0.0)
        cols.append(jnp.where(found, m, first))
    idx = jnp.concatenate(cols, axis=1).astype(jnp.int32)
    o_ref[...] = (idx + b * _N)[None]


def _ball(px, py, pz, cxt, cyt, czt):
    full2 = lambda shape: pl.BlockSpec(shape, lambda b, k: (0, 0))
    return pl.pallas_call(
        _ball_body,
        grid=(_B, _K // _KC),
        in_specs=[
            full2((_B, _N)), full2((_B, _N)), full2((_B, _N)),
            full2((_K, _B)), full2((_K, _B)), full2((_K, _B)),
        ],
        out_specs=pl.BlockSpec((1, _KC, _S), lambda b, k: (b, k, 0)),
        out_shape=jax.ShapeDtypeStruct((_B, _K, _S), jnp.int32),
    )(px, py, pz, cxt, cyt, czt)


def _sc_gather(table, idx):
    tot = idx.shape[0]
    per_w = tot // _NW
    nch = per_w // _CH
    mesh = plsc.VectorSubcoreMesh(core_axis_name="c", subcore_axis_name="s")

    @functools.partial(
        pl.kernel,
        mesh=mesh,
        out_type=jax.ShapeDtypeStruct((tot, _CP), jnp.float32),
        scratch_types=[
            pltpu.VMEM((_CH,), jnp.int32),
            pltpu.VMEM((_CH, _CP), jnp.float32),
            pltpu.SemaphoreType.DMA,
        ],
    )
    def gk(table_hbm, idx_hbm, out_hbm, idx_v, rows_v, sem):
        wid = lax.axis_index("s") * 2 + lax.axis_index("c")
        w0 = wid * per_w

        def body(ch, carry):
            base = pl.multiple_of(w0 + ch * _CH, _CH)
            pltpu.sync_copy(idx_hbm.at[pl.ds(base, _CH)], idx_v)
            pltpu.async_copy(table_hbm.at[idx_v], rows_v, sem).wait()
            pltpu.sync_copy(rows_v, out_hbm.at[pl.ds(base, _CH)])
            return carry

        lax.fori_loop(0, nch, body, 0)

    return gk(table, idx)


def _mlp_body(g_ref, c_ref, w1_ref, wc_ref, w2_ref, b2_ref, w3_ref,
              b3_ref, o_ref):
    f32 = jnp.float32
    bias = jnp.dot(c_ref[...], wc_ref[...], preferred_element_type=f32)
    h = jnp.dot(g_ref[...], w1_ref[...], preferred_element_type=f32)
    h = h.reshape(_KC, _S, 128) + bias[:, None, :]
    h = jnp.maximum(h, 0.0).reshape(_KC * _S, 128)
    h = jnp.dot(h, w2_ref[...], preferred_element_type=f32) + b2_ref[0:1, :]
    h = jnp.maximum(h, 0.0)
    h = jnp.dot(h, w3_ref[...], preferred_element_type=f32) + b3_ref[0:1, :]
    h = jnp.maximum(h, 0.0)
    o_ref[...] = jnp.max(h.reshape(_KC, _S, 256), axis=1)


def _mlp(g, caug, w1p, wc, w2, b2t, w3, b3t):
    rows = _KC * _S
    nblk = (_B * _K) // _KC
    whole = lambda shape: pl.BlockSpec(shape, lambda i: (0, 0))
    return pl.pallas_call(
        _mlp_body,
        grid=(nblk,),
        in_specs=[
            pl.BlockSpec((rows, _CP), lambda i: (i, 0)),
            pl.BlockSpec((_KC, 8), lambda i: (i, 0)),
            whole((_CP, 128)),
            whole((8, 128)),
            whole((128, 256)),
            whole((8, 256)),
            whole((256, 256)),
            whole((8, 256)),
        ],
        out_specs=pl.BlockSpec((_KC, 256), lambda i: (i, 0)),
        out_shape=jax.ShapeDtypeStruct((_B * _K, 256), jnp.float32),
    )(g, caug, w1p, wc, w2, b2t, w3, b3t)


def kernel(x, W1, b1, W2, b2, W3, b3):
    px = x[:, :, 0]
    py = x[:, :, 1]
    pz = x[:, :, 2]

    cx, cy, cz = _fps(px, py, pz)

    idx = _ball(px, py, pz, cx.T, cy.T, cz.T)

    xpad = jnp.pad(x, ((0, 0), (0, 0), (0, _CP - _C)))
    table = xpad.reshape(_B * _N, _CP)
    g = _sc_gather(table, idx.reshape(-1))

    ones = jnp.ones((_B * _K, 1), jnp.float32)
    zeros4 = jnp.zeros((_B * _K, 4), jnp.float32)
    caug = jnp.concatenate(
        [cx.reshape(-1, 1), cy.reshape(-1, 1), cz.reshape(-1, 1), ones,
         zeros4], axis=1)
    w1p = jnp.concatenate([W1, jnp.zeros((_CP - _C, 128), jnp.float32)],
                          axis=0)
    wc = jnp.concatenate(
        [-W1[:3], b1[None, :], jnp.zeros((4, 128), jnp.float32)], axis=0)
    b2t = jnp.broadcast_to(b2[None, :], (8, 256))
    b3t = jnp.broadcast_to(b3[None, :], (8, 256))

    h = _mlp(g, caug, w1p, wc, W2, b2t, W3, b3t)

    centroid = jnp.stack([cx, cy, cz], axis=-1)
    return jnp.concatenate(
        [centroid, h.reshape(_B, _K, 256)], axis=2)


# speedup vs baseline: 19.4090x; 1.0964x over previous
import functools

import jax
import jax.numpy as jnp
from jax import lax
from jax.experimental import pallas as pl
from jax.experimental.pallas import tpu as pltpu
from jax.experimental.pallas import tpu_sc as plsc

_B = 8
_N = 4096
_C = 35
_K = 512
_S = 32
_R2 = 0.25 * 0.25
_CP = 128
_KC = 128
_NW = 32
_CH = 128


def _fps_body(px_ref, py_ref, pz_ref, ocx_ref, ocy_ref, ocz_ref):
    px = px_ref[...]
    py = py_ref[...]
    pz = pz_ref[...]
    lane_n = lax.broadcasted_iota(jnp.int32, (_B, _N), 1)
    lane_k = lax.broadcasted_iota(jnp.int32, (_B, _K), 1)

    lx = px[:, 0:1]
    ly = py[:, 0:1]
    lz = pz[:, 0:1]
    zero_k = jnp.zeros((_B, _K), jnp.float32)
    acx = jnp.where(lane_k == 0, lx, zero_k)
    acy = jnp.where(lane_k == 0, ly, zero_k)
    acz = jnp.where(lane_k == 0, lz, zero_k)
    dists = jnp.full((_B, _N), jnp.inf, jnp.float32)

    def body(i, st):
        dists, lx, ly, lz, acx, acy, acz = st
        d = (px - lx) ** 2 + (py - ly) ** 2 + (pz - lz) ** 2
        dists = jnp.minimum(dists, d)
        nxt = jnp.argmax(dists, axis=1).astype(jnp.int32)
        onehot = lane_n == nxt[:, None]
        lx = jnp.sum(jnp.where(onehot, px, 0.0), axis=1, keepdims=True)
        ly = jnp.sum(jnp.where(onehot, py, 0.0), axis=1, keepdims=True)
        lz = jnp.sum(jnp.where(onehot, pz, 0.0), axis=1, keepdims=True)
        sel = lane_k == i
        acx = jnp.where(sel, lx, acx)
        acy = jnp.where(sel, ly, acy)
        acz = jnp.where(sel, lz, acz)
        return dists, lx, ly, lz, acx, acy, acz

    st = (dists, lx, ly, lz, acx, acy, acz)
    st = lax.fori_loop(1, _K, body, st)
    _, _, _, _, acx, acy, acz = st
    ocx_ref[...] = acx
    ocy_ref[...] = acy
    ocz_ref[...] = acz


def _fps(px, py, pz):
    out = [jax.ShapeDtypeStruct((_B, _K), jnp.float32)] * 3
    return pl.pallas_call(_fps_body, out_shape=out)(px, py, pz)


def _ball_body(b0, px_ref, py_ref, pz_ref, cx_ref, cy_ref, cz_ref, o_ref):
    b = pl.program_id(0) + b0
    kb = pl.program_id(1)
    rsel = lax.broadcasted_iota(jnp.int32, (_B, _N), 0) == b
    pick_row = lambda r: jnp.sum(jnp.where(rsel, r[...], 0.0), axis=0,
                                 keepdims=True)
    pxr = pick_row(px_ref)
    pyr = pick_row(py_ref)
    pzr = pick_row(pz_ref)
    k0 = kb * _KC
    csel = lax.broadcasted_iota(jnp.int32, (_KC, _B), 1) == b
    pick_col = lambda r: jnp.sum(
        jnp.where(csel, r[pl.ds(k0, _KC), :], 0.0), axis=1, keepdims=True)
    cxc = pick_col(cx_ref)
    cyc = pick_col(cy_ref)
    czc = pick_col(cz_ref)

    cn = cxc * cxc + cyc * cyc + czc * czc
    pn = pxr * pxr + pyr * pyr + pzr * pzr
    cmat = jnp.concatenate(
        [cxc, cyc, czc, jnp.zeros((_KC, 5), jnp.float32)], axis=1)
    pmat = jnp.concatenate(
        [pxr, pyr, pzr, jnp.zeros((5, _N), jnp.float32)], axis=0)
    dot = jnp.dot(cmat, pmat, preferred_element_type=jnp.float32)
    d2 = cn + pn - 2.0 * dot
    mask = d2 < _R2

    iota_f = lax.broadcasted_iota(jnp.int32, (_KC, _N), 1).astype(jnp.float32)
    bigf = jnp.float32(_N)
    vals = jnp.where(mask, iota_f, bigf)

    cols = []
    first = None
    m = jnp.full((_KC, 1), -1.0, jnp.float32)
    for s in range(_S):
        masked = jnp.where(vals > m, vals, bigf)
        m = jnp.min(masked, axis=1, keepdims=True)
        found = m < bigf
        if s == 0:
            first = jnp.where(found, m, 0.0)
        cols.append(jnp.where(found, m, first))
    idx = jnp.concatenate(cols, axis=1).astype(jnp.int32)
    o_ref[...] = (idx + b * _N)[None]


def _ball(px, py, pz, cxt, cyt, czt, b0, nb):
    full2 = lambda shape: pl.BlockSpec(shape, lambda b, k: (0, 0))
    return pl.pallas_call(
        functools.partial(_ball_body, b0),
        grid=(nb, _K // _KC),
        in_specs=[
            full2((_B, _N)), full2((_B, _N)), full2((_B, _N)),
            full2((_K, _B)), full2((_K, _B)), full2((_K, _B)),
        ],
        out_specs=pl.BlockSpec((1, _KC, _S), lambda b, k: (b, k, 0)),
        out_shape=jax.ShapeDtypeStruct((nb, _K, _S), jnp.int32),
    )(px, py, pz, cxt, cyt, czt)


def _sc_gather(table, idx):
    tot = idx.shape[0]
    per_w = tot // _NW
    nch = per_w // _CH
    mesh = plsc.VectorSubcoreMesh(core_axis_name="c", subcore_axis_name="s")

    @functools.partial(
        pl.kernel,
        mesh=mesh,
        out_type=jax.ShapeDtypeStruct((tot, _CP), jnp.float32),
        scratch_types=[
            pltpu.VMEM((_CH,), jnp.int32),
            pltpu.VMEM((_CH, _CP), jnp.float32),
            pltpu.SemaphoreType.DMA,
        ],
    )
    def gk(table_hbm, idx_hbm, out_hbm, idx_v, rows_v, sem):
        wid = lax.axis_index("s") * 2 + lax.axis_index("c")
        w0 = wid * per_w

        def body(ch, carry):
            base = pl.multiple_of(w0 + ch * _CH, _CH)
            pltpu.sync_copy(idx_hbm.at[pl.ds(base, _CH)], idx_v)
            pltpu.async_copy(table_hbm.at[idx_v], rows_v, sem).wait()
            pltpu.sync_copy(rows_v, out_hbm.at[pl.ds(base, _CH)])
            return carry

        lax.fori_loop(0, nch, body, 0)

    return gk(table, idx)


def _mlp_body(g_ref, c_ref, w1_ref, wc_ref, w2_ref, b2_ref, w3_ref,
              b3_ref, o_ref):
    f32 = jnp.float32
    bias = jnp.dot(c_ref[...], wc_ref[...], preferred_element_type=f32)
    h = jnp.dot(g_ref[...], w1_ref[...], preferred_element_type=f32)
    h = h.reshape(_KC, _S, 128) + bias[:, None, :]
    h = jnp.maximum(h, 0.0).reshape(_KC * _S, 128)
    h = jnp.dot(h, w2_ref[...], preferred_element_type=f32) + b2_ref[0:1, :]
    h = jnp.maximum(h, 0.0)
    h = jnp.dot(h, w3_ref[...], preferred_element_type=f32) + b3_ref[0:1, :]
    h = jnp.maximum(h, 0.0)
    o_ref[...] = jnp.max(h.reshape(_KC, _S, 256), axis=1)


def _mlp(g, caug, w1p, wc, w2, b2t, w3, b3t):
    rows = _KC * _S
    nblk = caug.shape[0] // _KC
    whole = lambda shape: pl.BlockSpec(shape, lambda i: (0, 0))
    return pl.pallas_call(
        _mlp_body,
        grid=(nblk,),
        in_specs=[
            pl.BlockSpec((rows, _CP), lambda i: (i, 0)),
            pl.BlockSpec((_KC, 8), lambda i: (i, 0)),
            whole((_CP, 128)),
            whole((8, 128)),
            whole((128, 256)),
            whole((8, 256)),
            whole((256, 256)),
            whole((8, 256)),
        ],
        out_specs=pl.BlockSpec((_KC, 256), lambda i: (i, 0)),
        out_shape=jax.ShapeDtypeStruct((caug.shape[0], 256), jnp.float32),
    )(g, caug, w1p, wc, w2, b2t, w3, b3t)


def kernel(x, W1, b1, W2, b2, W3, b3):
    px = x[:, :, 0]
    py = x[:, :, 1]
    pz = x[:, :, 2]

    cx, cy, cz = _fps(px, py, pz)

    xpad = jnp.pad(x, ((0, 0), (0, 0), (0, _CP - _C)))
    table = xpad.reshape(_B * _N, _CP)

    ones = jnp.ones((_B * _K, 1), jnp.float32)
    zeros4 = jnp.zeros((_B * _K, 4), jnp.float32)
    caug = jnp.concatenate(
        [cx.reshape(-1, 1), cy.reshape(-1, 1), cz.reshape(-1, 1), ones,
         zeros4], axis=1)
    w1p = jnp.concatenate([W1, jnp.zeros((_CP - _C, 128), jnp.float32)],
                          axis=0)
    wc = jnp.concatenate(
        [-W1[:3], b1[None, :], jnp.zeros((4, 128), jnp.float32)], axis=0)
    b2t = jnp.broadcast_to(b2[None, :], (8, 256))
    b3t = jnp.broadcast_to(b3[None, :], (8, 256))

    nb = _B // 2
    cxt, cyt, czt = cx.T, cy.T, cz.T
    hs = []
    for b0 in (0, nb):
        idxh = _ball(px, py, pz, cxt, cyt, czt, b0, nb)
        gh = _sc_gather(table, idxh.reshape(-1))
        ch = caug[b0 * _K:(b0 + nb) * _K]
        hs.append(_mlp(gh, ch, w1p, wc, W2, b2t, W3, b3t))
    h = jnp.concatenate(hs, axis=0)

    centroid = jnp.stack([cx, cy, cz], axis=-1)
    return jnp.concatenate(
        [centroid, h.reshape(_B, _K, 256)], axis=2)
